# baseline (speedup 1.0000x reference)
import dataclasses
import functools

import jax
import jax.numpy as jnp
from jax.experimental import pallas as pl
from jax.experimental.pallas import tpu as pltpu
from jax.experimental.pallas import tpu_sc as plsc

VOCAB = 100000
EMBED = 64
WINDOW = 5
HIDDEN = 128
OUT = 1000
BATCH = 16384
CONCAT = WINDOW * EMBED
WIDE = 2 * EMBED

CHUNKS = (8192, 8192)
OFFS = (0, 8192)

GATHER_WINDOW = 256
BATCH_TILE = 1024

PAD_ROWS = 2000


def _pad_body(t_ref, o_ref):
    o_ref[:, :EMBED] = t_ref[...]


def _tc_pad(table):
    return pl.pallas_call(
        _pad_body,
        grid=(VOCAB // PAD_ROWS,),
        in_specs=[pl.BlockSpec((PAD_ROWS, EMBED), lambda i: (i, 0))],
        out_specs=pl.BlockSpec((PAD_ROWS, WIDE), lambda i: (i, 0)),
        out_shape=jax.ShapeDtypeStruct((VOCAB, WIDE), jnp.float32),
    )(table)


def _sc_gather(table_pad, idx_flat, cb):
    mesh = plsc.VectorSubcoreMesh(core_axis_name="core", subcore_axis_name="subcore")
    n = cb * WINDOW

    @functools.partial(
        pl.kernel,
        out_type=jax.ShapeDtypeStruct((n, WIDE), table_pad.dtype),
        mesh=mesh,
    )
    def gather_kernel(table_hbm, idx_hbm, out_hbm):
        def body(idx_vmem, out_vmem):
            pltpu.sync_copy(table_hbm.at[idx_vmem], out_vmem)

        pltpu.emit_pipeline(
            body,
            grid=(n // GATHER_WINDOW,),
            in_specs=[
                pl.BlockSpec((GATHER_WINDOW,), index_map=lambda i: (i,))
            ],
            out_specs=[
                pl.BlockSpec((GATHER_WINDOW, WIDE), index_map=lambda i: (i, 0))
            ],
            core_axis_name=("core", "subcore"),
            dimension_semantics=(pltpu.PARALLEL,),
        )(idx_hbm, out_hbm)

    return gather_kernel(table_pad, idx_flat)


def _mlp_compute(wides, w1_ref, b1_ref, w2_ref, b2t_ref, out_ref):
    acc = b1_ref[...].astype(jnp.float32)
    for w in range(WINDOW):
        acc = acc + jax.lax.dot_general(
            wides[w][:, :EMBED], w1_ref[w * EMBED:(w + 1) * EMBED, :],
            (((1,), (0,)), ((), ())),
            precision=jax.lax.Precision.DEFAULT,
            preferred_element_type=jnp.float32,
        )
    h = jnp.tanh(acc)
    ot = jax.lax.dot_general(
        w2_ref[...], h,
        (((0,), (1,)), ((), ())),
        precision=jax.lax.Precision.DEFAULT,
        preferred_element_type=jnp.float32,
    ) + b2t_ref[...]
    m = jnp.max(ot, axis=0, keepdims=True)
    e = jnp.exp(ot - m)
    out_ref[...] = e / jnp.sum(e, axis=0, keepdims=True)


def _mlp_body(w0, w1, w2, w3, w4, w1_ref, b1_ref, w2_ref, b2t_ref, out_ref):
    _mlp_compute((w0, w1, w2, w3, w4), w1_ref, b1_ref, w2_ref, b2t_ref, out_ref)


def _mlp_body_alias(prev_ref, w0, w1, w2, w3, w4, w1_ref, b1_ref, w2_ref,
                    b2t_ref, out_ref):
    del prev_ref
    _mlp_compute((w0, w1, w2, w3, w4), w1_ref, b1_ref, w2_ref, b2t_ref, out_ref)


def _tc_mlp_chunk(off, cb, prev, wide_c, W1, b1, W2, b2):
    nbc = cb // BATCH_TILE
    wide_spec = lambda w: pl.BlockSpec(
        (BATCH_TILE, WIDE), functools.partial(lambda w, i: (w * nbc + i, 0), w))
    weight_specs = [
        pl.BlockSpec((CONCAT, HIDDEN), lambda i: (0, 0)),
        pl.BlockSpec((1, HIDDEN), lambda i: (0, 0)),
        pl.BlockSpec((HIDDEN, OUT), lambda i: (0, 0)),
        pl.BlockSpec((OUT, 1), lambda i: (0, 0)),
    ]
    ob = off // BATCH_TILE
    out_spec = pl.BlockSpec(
        (OUT, BATCH_TILE), functools.partial(lambda ob, i: (0, ob + i), ob))
    wide_args = (wide_c,) * WINDOW
    weight_args = (W1, b1.reshape(1, HIDDEN), W2, b2.reshape(OUT, 1))
    if prev is None:
        return pl.pallas_call(
            _mlp_body,
            grid=(nbc,),
            in_specs=[wide_spec(w) for w in range(WINDOW)] + weight_specs,
            out_specs=out_spec,
            out_shape=jax.ShapeDtypeStruct((OUT, BATCH), jnp.float32),
        )(*wide_args, *weight_args)
    return pl.pallas_call(
        _mlp_body_alias,
        grid=(nbc,),
        in_specs=[pl.BlockSpec(memory_space=pl.ANY)]
        + [wide_spec(w) for w in range(WINDOW)] + weight_specs,
        out_specs=out_spec,
        out_shape=jax.ShapeDtypeStruct((OUT, BATCH), jnp.float32),
        input_output_aliases={0: 0},
    )(prev, *wide_args, *weight_args)


def kernel(x, table, W1, b1, W2, b2):
    table_pad = _tc_pad(table)
    xt = x.T
    out = None
    for off, cb in zip(OFFS, CHUNKS):
        idx_c = xt[:, off:off + cb].reshape(-1)
        wide_c = _sc_gather(table_pad, idx_c, cb)
        out = _tc_mlp_chunk(off, cb, out, wide_c, W1, b1, W2, b2)
    return out.T


# speedup vs baseline: 1.2214x; 1.2214x over previous
import dataclasses
import functools

import jax
import jax.numpy as jnp
from jax.experimental import pallas as pl
from jax.experimental.pallas import tpu as pltpu
from jax.experimental.pallas import tpu_sc as plsc

VOCAB = 100000
EMBED = 64
WINDOW = 5
HIDDEN = 128
OUT = 1000
BATCH = 16384
CONCAT = WINDOW * EMBED
WIDE = 2 * EMBED

CHUNKS = (8192, 8192)
OFFS = (0, 8192)

GATHER_WINDOW = 256
BATCH_TILE = 1024

def _sc_gather(table_pad, idx_flat, cb):
    mesh = plsc.VectorSubcoreMesh(core_axis_name="core", subcore_axis_name="subcore")
    n = cb * WINDOW

    @functools.partial(
        pl.kernel,
        out_type=jax.ShapeDtypeStruct((n, WIDE), table_pad.dtype),
        mesh=mesh,
    )
    def gather_kernel(table_hbm, idx_hbm, out_hbm):
        def body(idx_vmem, out_vmem):
            pltpu.sync_copy(table_hbm.at[idx_vmem], out_vmem)

        pltpu.emit_pipeline(
            body,
            grid=(n // GATHER_WINDOW,),
            in_specs=[
                pl.BlockSpec((GATHER_WINDOW,), index_map=lambda i: (i,))
            ],
            out_specs=[
                pl.BlockSpec((GATHER_WINDOW, WIDE), index_map=lambda i: (i, 0))
            ],
            core_axis_name=("core", "subcore"),
            dimension_semantics=(pltpu.PARALLEL,),
        )(idx_hbm, out_hbm)

    return gather_kernel(table_pad, idx_flat)


def _mlp_compute(wides, w1_ref, b1_ref, w2_ref, b2t_ref, out_ref):
    acc = b1_ref[...].astype(jnp.float32)
    for w in range(WINDOW):
        acc = acc + jax.lax.dot_general(
            wides[w][:, :EMBED], w1_ref[w * EMBED:(w + 1) * EMBED, :],
            (((1,), (0,)), ((), ())),
            precision=jax.lax.Precision.DEFAULT,
            preferred_element_type=jnp.float32,
        )
    h = jnp.tanh(acc)
    ot = jax.lax.dot_general(
        w2_ref[...], h,
        (((0,), (1,)), ((), ())),
        precision=jax.lax.Precision.DEFAULT,
        preferred_element_type=jnp.float32,
    ) + b2t_ref[...]
    m = jnp.max(ot, axis=0, keepdims=True)
    e = jnp.exp(ot - m)
    out_ref[...] = e / jnp.sum(e, axis=0, keepdims=True)


def _mlp_body(w0, w1, w2, w3, w4, w1_ref, b1_ref, w2_ref, b2t_ref, out_ref):
    _mlp_compute((w0, w1, w2, w3, w4), w1_ref, b1_ref, w2_ref, b2t_ref, out_ref)


def _mlp_body_alias(prev_ref, w0, w1, w2, w3, w4, w1_ref, b1_ref, w2_ref,
                    b2t_ref, out_ref):
    del prev_ref
    _mlp_compute((w0, w1, w2, w3, w4), w1_ref, b1_ref, w2_ref, b2t_ref, out_ref)


def _tc_mlp_chunk(off, cb, prev, wide_c, W1, b1, W2, b2):
    nbc = cb // BATCH_TILE
    wide_spec = lambda w: pl.BlockSpec(
        (BATCH_TILE, WIDE), functools.partial(lambda w, i: (w * nbc + i, 0), w))
    weight_specs = [
        pl.BlockSpec((CONCAT, HIDDEN), lambda i: (0, 0)),
        pl.BlockSpec((1, HIDDEN), lambda i: (0, 0)),
        pl.BlockSpec((HIDDEN, OUT), lambda i: (0, 0)),
        pl.BlockSpec((OUT, 1), lambda i: (0, 0)),
    ]
    ob = off // BATCH_TILE
    out_spec = pl.BlockSpec(
        (OUT, BATCH_TILE), functools.partial(lambda ob, i: (0, ob + i), ob))
    wide_args = (wide_c,) * WINDOW
    weight_args = (W1, b1.reshape(1, HIDDEN), W2, b2.reshape(OUT, 1))
    if prev is None:
        return pl.pallas_call(
            _mlp_body,
            grid=(nbc,),
            in_specs=[wide_spec(w) for w in range(WINDOW)] + weight_specs,
            out_specs=out_spec,
            out_shape=jax.ShapeDtypeStruct((OUT, BATCH), jnp.float32),
        )(*wide_args, *weight_args)
    return pl.pallas_call(
        _mlp_body_alias,
        grid=(nbc,),
        in_specs=[pl.BlockSpec(memory_space=pl.ANY)]
        + [wide_spec(w) for w in range(WINDOW)] + weight_specs,
        out_specs=out_spec,
        out_shape=jax.ShapeDtypeStruct((OUT, BATCH), jnp.float32),
        input_output_aliases={0: 0},
    )(prev, *wide_args, *weight_args)


def kernel(x, table, W1, b1, W2, b2):
    table_pad = jnp.pad(table, ((0, 0), (0, WIDE - EMBED)))
    xt = x.T
    out = None
    for off, cb in zip(OFFS, CHUNKS):
        idx_c = xt[:, off:off + cb].reshape(-1)
        wide_c = _sc_gather(table_pad, idx_c, cb)
        out = _tc_mlp_chunk(off, cb, out, wide_c, W1, b1, W2, b2)
    return out.T
